# baseline (speedup 1.0000x reference)
import functools

import jax
import jax.numpy as jnp
from jax import lax
from jax.experimental import pallas as pl
from jax.experimental.pallas import tpu as pltpu
from jax.experimental.pallas import tpu_sc as plsc

N = 10000
D = 128
HALF = 64
E = 320000
P = 100000

NT = 16
NC = 2
CH = 96
NCH = -(-E // (NT * CH))
EP = NCH * NT * CH
EPT = NCH * CH
NP = N + 8
RA = (N // NT) // 8 * 8
NREM = N - RA * NT
BP = 3200
PP = NT * NC * BP


def _elu(x):
    return jnp.where(x > 0, x, jnp.exp(jnp.minimum(x, 0.0)) - 1.0)


def _tc0_body(x_ref, w_ref, b_ref, o_ref):
    o_ref[...] = _elu(
        jnp.dot(x_ref[...], w_ref[...], preferred_element_type=jnp.float32)
        + b_ref[...])


def _node_emb(x, W, b):
    R = 2000
    return pl.pallas_call(
        _tc0_body,
        grid=(N // R,),
        in_specs=[pl.BlockSpec((R, D), lambda i: (i, 0)),
                  pl.BlockSpec((D, D), lambda i: (0, 0)),
                  pl.BlockSpec((1, D), lambda i: (0, 0))],
        out_specs=pl.BlockSpec((R, D), lambda i: (i, 0)),
        out_shape=jax.ShapeDtypeStruct((N, D), jnp.float32),
    )(x, W, b.reshape(1, D))


def _tc1_body(h_ref, w_ref, b_ref, w1_ref, b1_ref, w2_ref, sems_ref, wp_ref):
    ws = []
    for i in range(4):
        x = h_ref[i]
        sem = _elu(
            jnp.dot(x, w_ref[i], preferred_element_type=jnp.float32) + b_ref[i])
        sems_ref[i] = sem
        t = jnp.tanh(
            jnp.dot(sem, w1_ref[i // 2], preferred_element_type=jnp.float32)
            + b1_ref[i // 2])
        ws.append(jnp.sum(t * w2_ref[i // 2]))
    wp_ref[...] = jnp.stack(ws)[None, None, :]


def _sems_and_w(hsum, Weff, beff, W1, b1, W2):
    R = 2000
    G = N // R
    return pl.pallas_call(
        _tc1_body,
        grid=(G,),
        in_specs=[pl.BlockSpec((4, R, D), lambda i: (0, i, 0)),
                  pl.BlockSpec((4, D, D), lambda i: (0, 0, 0)),
                  pl.BlockSpec((4, 1, D), lambda i: (0, 0, 0)),
                  pl.BlockSpec((2, D, HALF), lambda i: (0, 0, 0)),
                  pl.BlockSpec((2, 1, HALF), lambda i: (0, 0, 0)),
                  pl.BlockSpec((2, 1, HALF), lambda i: (0, 0, 0))],
        out_specs=[pl.BlockSpec((4, R, D), lambda i: (0, i, 0)),
                   pl.BlockSpec((1, 1, 4), lambda i: (i, 0, 0))],
        out_shape=[jax.ShapeDtypeStruct((4, N, D), jnp.float32),
                   jax.ShapeDtypeStruct((G, 1, 4), jnp.float32)],
    )(hsum, Weff, beff, W1, b1, W2)


def _tc2_body(s_ref, beta_ref, fcw_ref, fcb_ref, pw_ref, ne2_ref, uv_ref):
    emb_in = beta_ref[0, 0] * s_ref[0] + beta_ref[0, 1] * s_ref[1]
    emb_out = beta_ref[0, 2] * s_ref[2] + beta_ref[0, 3] * s_ref[3]
    ne2 = _elu(
        jnp.dot(emb_in, fcw_ref[0], preferred_element_type=jnp.float32)
        + jnp.dot(emb_out, fcw_ref[1], preferred_element_type=jnp.float32)
        + fcb_ref[...])
    ne2_ref[...] = ne2
    uv_ref[...] = jnp.dot(ne2, pw_ref[...], preferred_element_type=jnp.float32)


def _fuse(sems, beta, fcW, fcb, predWcat):
    R = 2000
    return pl.pallas_call(
        _tc2_body,
        grid=(N // R,),
        in_specs=[pl.BlockSpec((4, R, D), lambda i: (0, i, 0)),
                  pl.BlockSpec(memory_space=pltpu.SMEM),
                  pl.BlockSpec((2, D, D), lambda i: (0, 0, 0)),
                  pl.BlockSpec((1, D), lambda i: (0, 0)),
                  pl.BlockSpec((D, 4), lambda i: (0, 0))],
        out_specs=[pl.BlockSpec((R, D), lambda i: (i, 0)),
                   pl.BlockSpec((R, 4), lambda i: (i, 0))],
        out_shape=[jax.ShapeDtypeStruct((N, D), jnp.float32),
                   jax.ShapeDtypeStruct((N, 4), jnp.float32)],
    )(sems, beta, fcW, fcb, predWcat)


NQ = 8
NRB = 4
NGRP = (NCH - 1) // NQ


def _agg(node_emb_p, srcs, dsts):
    mesh = plsc.VectorSubcoreMesh(core_axis_name="c", subcore_axis_name="s")

    @functools.partial(
        pl.kernel, mesh=mesh,
        out_type=jax.ShapeDtypeStruct((4, N, D), jnp.float32),
        scratch_types=[
            [pltpu.VMEM((CH,), jnp.int32) for _ in range(NQ)],
            [pltpu.VMEM((CH,), jnp.int32) for _ in range(NQ)],
            [pltpu.VMEM((CH, D), jnp.float32) for _ in range(NRB)],
            pltpu.VMEM_SHARED((N, D), jnp.float32),
            [pltpu.SemaphoreType.DMA for _ in range(NQ)],
            [pltpu.SemaphoreType.DMA for _ in range(NRB)],
            [pltpu.SemaphoreType.DMA for _ in range(NRB)],
        ],
        compiler_params=pltpu.CompilerParams(use_tc_tiling_on_sc=False, needs_layout_passes=False),
    )
    def k(ne_hbm, srcs_hbm, dsts_hbm, out_hbm,
          sidx, didx, rows, acc, isem, gsem, ssem):
        c = lax.axis_index("c")
        s = lax.axis_index("s")

        def idx_cp(base, q):
            return (pltpu.make_async_copy(srcs_hbm.at[pl.ds(base, CH)],
                                          sidx[q], isem[q]),
                    pltpu.make_async_copy(dsts_hbm.at[pl.ds(base, CH)],
                                          didx[q], isem[q]))

        def idx_start(base, q):
            a, bb = idx_cp(base, q)
            a.start()
            bb.start()

        def idx_wait(q):
            a, bb = idx_cp(0, q)
            a.wait()
            bb.wait()

        def gather(q, b):
            return pltpu.make_async_copy(ne_hbm.at[sidx[q]], rows[b], gsem[b])

        def scat(q, b):
            return pltpu.make_async_copy(rows[b], acc.at[didx[q]], ssem[b])

        for j in range(2):
            l = 2 * c + j
            ebase = l * EP + s * EPT
            pltpu.sync_copy(ne_hbm.at[pl.ds(s * RA, RA)],
                            acc.at[pl.ds(s * RA, RA)])

            @pl.when(s == 0)
            def _():
                pltpu.sync_copy(ne_hbm.at[pl.ds(RA * NT, NREM)],
                                acc.at[pl.ds(RA * NT, NREM)])

            plsc.subcore_barrier()

            for q in range(6):
                idx_start(ebase + q * CH, q)
            idx_wait(0)
            gather(0, 0).start()
            idx_wait(1)
            gather(1, 1).start()

            def body(k_, _):
                for u in range(NQ):
                    g = k_ * NQ + u
                    b = u % NRB
                    gather(u, b).wait()

                    @pl.when(g >= 2)
                    def _():
                        scat((u + 2) % NQ, (u + 2) % NRB).wait()

                    @pl.when(g + 6 < NCH)
                    def _():
                        idx_start(ebase + (g + 6) * CH, (u + 6) % NQ)

                    @pl.when(g + 2 < NCH)
                    def _():
                        idx_wait((u + 2) % NQ)
                        gather((u + 2) % NQ, (u + 2) % NRB).start()

                    scat(u, b).start(add=True)
                return ()

            lax.fori_loop(0, NGRP, body, ())

            gather(0, 0).wait()
            scat(2, 2).wait()
            scat(0, 0).start(add=True)
            scat(3, 3).wait()
            scat(0, 0).wait()

            plsc.subcore_barrier()
            pltpu.sync_copy(acc.at[pl.ds(s * RA, RA)],
                            out_hbm.at[l, pl.ds(s * RA, RA)])

            @pl.when(s == 0)
            def _():
                pltpu.sync_copy(acc.at[pl.ds(RA * NT, NREM)],
                                out_hbm.at[l, pl.ds(RA * NT, NREM)])

            plsc.subcore_barrier()

    return k(node_emb_p, srcs, dsts)


def _pair_logits(UVflat, eidx_flat):
    mesh = plsc.VectorSubcoreMesh(core_axis_name="c", subcore_axis_name="s")

    @functools.partial(
        pl.kernel, mesh=mesh,
        out_type=jax.ShapeDtypeStruct((2 * PP,), jnp.float32),
        scratch_types=[
            pltpu.VMEM((4 * N,), jnp.float32),
            pltpu.VMEM((BP,), jnp.int32),
            pltpu.VMEM((BP,), jnp.int32),
            pltpu.VMEM((BP,), jnp.float32),
            pltpu.VMEM((BP,), jnp.float32),
        ],
        compiler_params=pltpu.CompilerParams(use_tc_tiling_on_sc=False, needs_layout_passes=False),
    )
    def k(uv_hbm, eidx_hbm, out_hbm, uv_v, i0_v, i1_v, o0_v, o1_v):
        c = lax.axis_index("c")
        s = lax.axis_index("s")
        wid = s * NC + c
        base = wid * BP
        pltpu.sync_copy(uv_hbm, uv_v)
        pltpu.sync_copy(eidx_hbm.at[pl.ds(base, BP)], i0_v)
        pltpu.sync_copy(eidx_hbm.at[pl.ds(PP + base, BP)], i1_v)

        def body(g, _):
            sl = pl.ds(g * 16, 16)
            i0 = i0_v[sl] * 4
            i1 = i1_v[sl] * 4
            u0 = plsc.load_gather(uv_v, [i0])
            v0 = plsc.load_gather(uv_v, [i1 + 2])
            u1 = plsc.load_gather(uv_v, [i0 + 1])
            v1 = plsc.load_gather(uv_v, [i1 + 3])
            o0_v[sl] = u0 + v0
            o1_v[sl] = u1 + v1
            return ()

        lax.fori_loop(0, BP // 16, body, ())
        pltpu.sync_copy(o0_v, out_hbm.at[pl.ds(base, BP)])
        pltpu.sync_copy(o1_v, out_hbm.at[pl.ds(PP + base, BP)])

    return k(UVflat, eidx_flat)


def _edge_cs(edge_feat, fc_edge):
    emb = jax.nn.elu(jnp.squeeze(jnp.matmul(edge_feat, fc_edge), axis=1))
    r1, r2 = jnp.split(emb, 2, axis=-1)
    nrm = jnp.maximum(jnp.sqrt(r1 * r1 + r2 * r2), 1e-12)
    return r1 / nrm, r2 / nrm


def _step_mat(c, s, tag):
    if tag == "in":
        return (c, -s, s * c, c - s * s)
    return (c, s, -c * s, c - s * s)


def _compose(mb, ma):
    b00, b01, b10, b11 = mb
    a00, a01, a10, a11 = ma
    return (b00 * a00 + b01 * a10, b00 * a01 + b01 * a11,
            b10 * a00 + b11 * a10, b10 * a01 + b11 * a11)


def _path_mat(c, s, path, tag):
    mats = {e: _step_mat(c[e - 1], s[e - 1], tag) for e in (1, 2)}
    seq = path if tag == "in" else list(reversed(path))
    M = mats[seq[0]]
    for e in seq[1:]:
        M = _compose(mats[e], M)
    return M


def _weff(M, W):
    m00, m01, m10, m11 = M
    Wt, Wb = W[:HALF], W[HALF:]
    we = m00[:, None] * Wt + m10[:, None] * Wb
    wo = m01[:, None] * Wt + m11[:, None] * Wb
    return jnp.stack([we, wo], axis=1).reshape(D, D)


def kernel(node_feat, fc_node_W, fc_node_b, edge_feat_in, edge_feat_out,
           fc_edge_in, fc_edge_out, gin_in_W, gin_in_b, sa_in_W1, sa_in_b1,
           sa_in_W2, gin_out_W, gin_out_b, sa_out_W1, sa_out_b1, sa_out_W2,
           fc_W, fc_b, pred_W, edge_index_in_0, edge_index_in_1,
           edge_index_out_0, edge_index_out_1, edge_indices):
    node_emb = _node_emb(node_feat, fc_node_W, fc_node_b)

    cin, sin = _edge_cs(edge_feat_in, fc_edge_in)
    cout, sout = _edge_cs(edge_feat_out, fc_edge_out)
    Ms = [_path_mat(cin, sin, [1, 2], "in"),
          _path_mat(cin, sin, [2, 1], "in"),
          _path_mat(cout, sout, [1, 2], "out"),
          _path_mat(cout, sout, [2, 1], "out")]
    Gw = [gin_in_W[0], gin_in_W[1], gin_out_W[0], gin_out_W[1]]
    Weff = jnp.stack([_weff(Ms[i], Gw[i]) for i in range(4)])
    beff = jnp.stack([gin_in_b, gin_out_b]).reshape(4, 1, D)
    W1 = jnp.stack([sa_in_W1, sa_out_W1])
    b1 = jnp.stack([sa_in_b1, sa_out_b1]).reshape(2, 1, HALF)
    W2 = jnp.stack([sa_in_W2, sa_out_W2]).reshape(2, 1, HALF)

    srcs2 = jnp.stack([edge_index_in_0[0], edge_index_in_1[0],
                       edge_index_out_0[0], edge_index_out_1[0]])
    dsts2 = jnp.stack([edge_index_in_0[1], edge_index_in_1[1],
                       edge_index_out_0[1], edge_index_out_1[1]])
    srcs = jnp.concatenate(
        [srcs2, jnp.full((4, EP - E), N, jnp.int32)], axis=1).reshape(-1)
    dsts = jnp.concatenate(
        [dsts2, jnp.zeros((4, EP - E), jnp.int32)], axis=1).reshape(-1)
    ne_p = jnp.concatenate(
        [node_emb, jnp.zeros((NP - N, D), jnp.float32)])

    hsum = _agg(ne_p, srcs, dsts)
    sems, wpart = _sems_and_w(hsum, Weff, beff, W1, b1, W2)

    wsum = jnp.sum(wpart.reshape(-1, 4), axis=0) / N
    beta = jnp.concatenate([jax.nn.softmax(wsum[:2]),
                            jax.nn.softmax(wsum[2:])]).reshape(1, 4)

    fcWs = jnp.stack([fc_W[:D], fc_W[D:]])
    predWcat = jnp.concatenate([pred_W[:D], pred_W[D:]], axis=1)
    node_emb2, UV = _fuse(sems, beta, fcWs, fc_b.reshape(1, D), predWcat)

    eidx_flat = jnp.pad(edge_indices, ((0, PP - P), (0, 0))).T.reshape(-1)
    lg = _pair_logits(UV.reshape(-1), eidx_flat)
    logits = lg.reshape(2, PP).T[:P]
    return (node_emb2, logits)


# speedup vs baseline: 1.4031x; 1.4031x over previous
import functools

import jax
import jax.numpy as jnp
from jax import lax
from jax.experimental import pallas as pl
from jax.experimental.pallas import tpu as pltpu
from jax.experimental.pallas import tpu_sc as plsc

N = 10000
D = 128
HALF = 64
E = 320000
P = 100000

NT = 16
NC = 2
CH = 128
ECH = (E // NT) // CH * CH
NCH = ECH // CH
EREM_CH = (E - ECH * NT) // CH
RA = (N // NT) // 8 * 8
NREM = N - RA * NT
BP = 3200
PP = NT * NC * BP


def _elu(x):
    return jnp.where(x > 0, x, jnp.exp(jnp.minimum(x, 0.0)) - 1.0)


def _tc0_body(x_ref, w_ref, b_ref, o_ref):
    o_ref[...] = _elu(
        jnp.dot(x_ref[...], w_ref[...], preferred_element_type=jnp.float32)
        + b_ref[...])


def _node_emb(x, W, b):
    R = 2000
    return pl.pallas_call(
        _tc0_body,
        grid=(N // R,),
        in_specs=[pl.BlockSpec((R, D), lambda i: (i, 0)),
                  pl.BlockSpec((D, D), lambda i: (0, 0)),
                  pl.BlockSpec((1, D), lambda i: (0, 0))],
        out_specs=pl.BlockSpec((R, D), lambda i: (i, 0)),
        out_shape=jax.ShapeDtypeStruct((N, D), jnp.float32),
    )(x, W, b.reshape(1, D))


def _tc1_body(h_ref, w_ref, b_ref, w1_ref, b1_ref, w2_ref, sems_ref, wp_ref):
    ws = []
    for i in range(4):
        x = h_ref[i]
        sem = _elu(
            jnp.dot(x, w_ref[i], preferred_element_type=jnp.float32) + b_ref[i])
        sems_ref[i] = sem
        t = jnp.tanh(
            jnp.dot(sem, w1_ref[i // 2], preferred_element_type=jnp.float32)
            + b1_ref[i // 2])
        ws.append(jnp.sum(t * w2_ref[i // 2]))
    wp_ref[...] = jnp.stack(ws)[None, None, :]


def _sems_and_w(hsum, Weff, beff, W1, b1, W2):
    R = 2000
    G = N // R
    return pl.pallas_call(
        _tc1_body,
        grid=(G,),
        in_specs=[pl.BlockSpec((4, R, D), lambda i: (0, i, 0)),
                  pl.BlockSpec((4, D, D), lambda i: (0, 0, 0)),
                  pl.BlockSpec((4, 1, D), lambda i: (0, 0, 0)),
                  pl.BlockSpec((2, D, HALF), lambda i: (0, 0, 0)),
                  pl.BlockSpec((2, 1, HALF), lambda i: (0, 0, 0)),
                  pl.BlockSpec((2, 1, HALF), lambda i: (0, 0, 0))],
        out_specs=[pl.BlockSpec((4, R, D), lambda i: (0, i, 0)),
                   pl.BlockSpec((1, 1, 4), lambda i: (i, 0, 0))],
        out_shape=[jax.ShapeDtypeStruct((4, N, D), jnp.float32),
                   jax.ShapeDtypeStruct((G, 1, 4), jnp.float32)],
    )(hsum, Weff, beff, W1, b1, W2)


def _tc2_body(s_ref, beta_ref, fcw_ref, fcb_ref, pw_ref, ne2_ref, uv_ref):
    emb_in = beta_ref[0, 0] * s_ref[0] + beta_ref[0, 1] * s_ref[1]
    emb_out = beta_ref[0, 2] * s_ref[2] + beta_ref[0, 3] * s_ref[3]
    ne2 = _elu(
        jnp.dot(emb_in, fcw_ref[0], preferred_element_type=jnp.float32)
        + jnp.dot(emb_out, fcw_ref[1], preferred_element_type=jnp.float32)
        + fcb_ref[...])
    ne2_ref[...] = ne2
    uv_ref[...] = jnp.dot(ne2, pw_ref[...], preferred_element_type=jnp.float32)


def _fuse(sems, beta, fcW, fcb, predWcat):
    R = 2000
    return pl.pallas_call(
        _tc2_body,
        grid=(N // R,),
        in_specs=[pl.BlockSpec((4, R, D), lambda i: (0, i, 0)),
                  pl.BlockSpec(memory_space=pltpu.SMEM),
                  pl.BlockSpec((2, D, D), lambda i: (0, 0, 0)),
                  pl.BlockSpec((1, D), lambda i: (0, 0)),
                  pl.BlockSpec((D, 4), lambda i: (0, 0))],
        out_specs=[pl.BlockSpec((R, D), lambda i: (i, 0)),
                   pl.BlockSpec((R, 4), lambda i: (i, 0))],
        out_shape=[jax.ShapeDtypeStruct((N, D), jnp.float32),
                   jax.ShapeDtypeStruct((N, 4), jnp.float32)],
    )(sems, beta, fcW, fcb, predWcat)


NQ = 6
NRB = 3


def _agg(node_emb, srcs, dsts):
    mesh = plsc.VectorSubcoreMesh(core_axis_name="c", subcore_axis_name="s")

    @functools.partial(
        pl.kernel, mesh=mesh,
        out_type=jax.ShapeDtypeStruct((4, N, D), jnp.float32),
        scratch_types=[
            [pltpu.VMEM((CH,), jnp.int32) for _ in range(NQ)],
            [pltpu.VMEM((CH,), jnp.int32) for _ in range(NQ)],
            [pltpu.VMEM((CH, D), jnp.float32) for _ in range(NRB)],
            pltpu.VMEM_SHARED((N, D), jnp.float32),
            [pltpu.SemaphoreType.DMA for _ in range(NQ)],
            [pltpu.SemaphoreType.DMA for _ in range(NRB)],
            [pltpu.SemaphoreType.DMA for _ in range(NRB)],
        ],
        compiler_params=pltpu.CompilerParams(use_tc_tiling_on_sc=False, needs_layout_passes=False),
    )
    def k(ne_hbm, srcs_hbm, dsts_hbm, out_hbm,
          sidx, didx, rows, acc, isem, gsem, ssem):
        c = lax.axis_index("c")
        s = lax.axis_index("s")

        def idx_cp(base, q):
            return (pltpu.make_async_copy(srcs_hbm.at[pl.ds(base, CH)],
                                          sidx[q], isem[q]),
                    pltpu.make_async_copy(dsts_hbm.at[pl.ds(base, CH)],
                                          didx[q], isem[q]))

        def idx_start(base, q):
            a, bb = idx_cp(base, q)
            a.start()
            bb.start()

        def idx_wait(q):
            a, bb = idx_cp(0, q)
            a.wait()
            bb.wait()

        def gather(q, b):
            return pltpu.make_async_copy(ne_hbm.at[sidx[q]], rows[b], gsem[b])

        def scat(q, b):
            return pltpu.make_async_copy(rows[b], acc.at[didx[q]], ssem[b])

        for j in range(2):
            l = 2 * c + j
            lbase = l * E
            ebase = lbase + s * ECH
            pltpu.sync_copy(ne_hbm.at[pl.ds(s * RA, RA)],
                            acc.at[pl.ds(s * RA, RA)])

            @pl.when(s == 0)
            def _():
                pltpu.sync_copy(ne_hbm.at[pl.ds(RA * NT, NREM)],
                                acc.at[pl.ds(RA * NT, NREM)])

            plsc.subcore_barrier()

            for q in range(4):
                idx_start(ebase + q * CH, q)
            idx_wait(0)
            gather(0, 0).start()
            idx_wait(1)
            gather(1, 1).start()

            def body(k_, _):
                for u in range(NQ):
                    g = k_ * NQ + u
                    b = u % NRB
                    gather(u, b).wait()

                    @pl.when(g >= 1)
                    def _():
                        scat((u + 2) % NQ, (u + 2) % NRB).wait()

                    @pl.when(g + 4 < NCH)
                    def _():
                        idx_start(ebase + (g + 4) * CH, (u + 4) % NQ)

                    @pl.when(g + 2 < NCH)
                    def _():
                        idx_wait((u + 2) % NQ)
                        gather((u + 2) % NQ, (u + 2) % NRB).start()

                    scat(u, b).start(add=True)
                return ()

            lax.fori_loop(0, NCH // NQ, body, ())
            scat(5, 2).wait()

            @pl.when(s < EREM_CH)
            def _():
                rbase = lbase + ECH * NT + s * CH
                idx_start(rbase, 0)
                idx_wait(0)
                pltpu.async_copy(ne_hbm.at[sidx[0]], rows[0], gsem[0]).wait()
                pltpu.sync_copy(rows[0], acc.at[didx[0]], add=True)

            plsc.subcore_barrier()
            pltpu.sync_copy(acc.at[pl.ds(s * RA, RA)],
                            out_hbm.at[l, pl.ds(s * RA, RA)])

            @pl.when(s == 0)
            def _():
                pltpu.sync_copy(acc.at[pl.ds(RA * NT, NREM)],
                                out_hbm.at[l, pl.ds(RA * NT, NREM)])

            plsc.subcore_barrier()

    return k(node_emb, srcs, dsts)


def _pair_logits(UVflat, eidx_flat):
    mesh = plsc.VectorSubcoreMesh(core_axis_name="c", subcore_axis_name="s")

    @functools.partial(
        pl.kernel, mesh=mesh,
        out_type=jax.ShapeDtypeStruct((2 * PP,), jnp.float32),
        scratch_types=[
            pltpu.VMEM((4 * N,), jnp.float32),
            pltpu.VMEM((BP,), jnp.int32),
            pltpu.VMEM((BP,), jnp.int32),
            pltpu.VMEM((BP,), jnp.float32),
            pltpu.VMEM((BP,), jnp.float32),
        ],
        compiler_params=pltpu.CompilerParams(use_tc_tiling_on_sc=False, needs_layout_passes=False),
    )
    def k(uv_hbm, eidx_hbm, out_hbm, uv_v, i0_v, i1_v, o0_v, o1_v):
        c = lax.axis_index("c")
        s = lax.axis_index("s")
        wid = s * NC + c
        base = wid * BP
        pltpu.sync_copy(uv_hbm, uv_v)
        pltpu.sync_copy(eidx_hbm.at[pl.ds(base, BP)], i0_v)
        pltpu.sync_copy(eidx_hbm.at[pl.ds(PP + base, BP)], i1_v)

        def body(g, _):
            sl = pl.ds(g * 16, 16)
            i0 = i0_v[sl] * 4
            i1 = i1_v[sl] * 4
            u0 = plsc.load_gather(uv_v, [i0])
            v0 = plsc.load_gather(uv_v, [i1 + 2])
            u1 = plsc.load_gather(uv_v, [i0 + 1])
            v1 = plsc.load_gather(uv_v, [i1 + 3])
            o0_v[sl] = u0 + v0
            o1_v[sl] = u1 + v1
            return ()

        lax.fori_loop(0, BP // 16, body, ())
        pltpu.sync_copy(o0_v, out_hbm.at[pl.ds(base, BP)])
        pltpu.sync_copy(o1_v, out_hbm.at[pl.ds(PP + base, BP)])

    return k(UVflat, eidx_flat)


def _edge_cs(edge_feat, fc_edge):
    emb = jax.nn.elu(jnp.squeeze(jnp.matmul(edge_feat, fc_edge), axis=1))
    r1, r2 = jnp.split(emb, 2, axis=-1)
    nrm = jnp.maximum(jnp.sqrt(r1 * r1 + r2 * r2), 1e-12)
    return r1 / nrm, r2 / nrm


def _step_mat(c, s, tag):
    if tag == "in":
        return (c, -s, s * c, c - s * s)
    return (c, s, -c * s, c - s * s)


def _compose(mb, ma):
    b00, b01, b10, b11 = mb
    a00, a01, a10, a11 = ma
    return (b00 * a00 + b01 * a10, b00 * a01 + b01 * a11,
            b10 * a00 + b11 * a10, b10 * a01 + b11 * a11)


def _path_mat(c, s, path, tag):
    mats = {e: _step_mat(c[e - 1], s[e - 1], tag) for e in (1, 2)}
    seq = path if tag == "in" else list(reversed(path))
    M = mats[seq[0]]
    for e in seq[1:]:
        M = _compose(mats[e], M)
    return M


def _weff(M, W):
    m00, m01, m10, m11 = M
    Wt, Wb = W[:HALF], W[HALF:]
    we = m00[:, None] * Wt + m10[:, None] * Wb
    wo = m01[:, None] * Wt + m11[:, None] * Wb
    return jnp.stack([we, wo], axis=1).reshape(D, D)


def kernel(node_feat, fc_node_W, fc_node_b, edge_feat_in, edge_feat_out,
           fc_edge_in, fc_edge_out, gin_in_W, gin_in_b, sa_in_W1, sa_in_b1,
           sa_in_W2, gin_out_W, gin_out_b, sa_out_W1, sa_out_b1, sa_out_W2,
           fc_W, fc_b, pred_W, edge_index_in_0, edge_index_in_1,
           edge_index_out_0, edge_index_out_1, edge_indices):
    node_emb = _node_emb(node_feat, fc_node_W, fc_node_b)

    cin, sin = _edge_cs(edge_feat_in, fc_edge_in)
    cout, sout = _edge_cs(edge_feat_out, fc_edge_out)
    Ms = [_path_mat(cin, sin, [1, 2], "in"),
          _path_mat(cin, sin, [2, 1], "in"),
          _path_mat(cout, sout, [1, 2], "out"),
          _path_mat(cout, sout, [2, 1], "out")]
    Gw = [gin_in_W[0], gin_in_W[1], gin_out_W[0], gin_out_W[1]]
    Weff = jnp.stack([_weff(Ms[i], Gw[i]) for i in range(4)])
    beff = jnp.stack([gin_in_b, gin_out_b]).reshape(4, 1, D)
    W1 = jnp.stack([sa_in_W1, sa_out_W1])
    b1 = jnp.stack([sa_in_b1, sa_out_b1]).reshape(2, 1, HALF)
    W2 = jnp.stack([sa_in_W2, sa_out_W2]).reshape(2, 1, HALF)

    srcs = jnp.concatenate([edge_index_in_0[0], edge_index_in_1[0],
                            edge_index_out_0[0], edge_index_out_1[0]])
    dsts = jnp.concatenate([edge_index_in_0[1], edge_index_in_1[1],
                            edge_index_out_0[1], edge_index_out_1[1]])

    hsum = _agg(node_emb, srcs, dsts)
    sems, wpart = _sems_and_w(hsum, Weff, beff, W1, b1, W2)

    wsum = jnp.sum(wpart.reshape(-1, 4), axis=0) / N
    beta = jnp.concatenate([jax.nn.softmax(wsum[:2]),
                            jax.nn.softmax(wsum[2:])]).reshape(1, 4)

    fcWs = jnp.stack([fc_W[:D], fc_W[D:]])
    predWcat = jnp.concatenate([pred_W[:D], pred_W[D:]], axis=1)
    node_emb2, UV = _fuse(sems, beta, fcWs, fc_b.reshape(1, D), predWcat)

    eidx_flat = jnp.pad(edge_indices, ((0, PP - P), (0, 0))).T.reshape(-1)
    lg = _pair_logits(UV.reshape(-1), eidx_flat)
    logits = lg.reshape(2, PP).T[:P]
    return (node_emb2, logits)


# speedup vs baseline: 1.4496x; 1.0332x over previous
import functools

import jax
import jax.numpy as jnp
from jax import lax
from jax.experimental import pallas as pl
from jax.experimental.pallas import tpu as pltpu
from jax.experimental.pallas import tpu_sc as plsc

N = 10000
D = 128
HALF = 64
E = 320000
P = 100000

NT = 16
NC = 2
CH = 128
ECH = (E // NT) // CH * CH
NCH = ECH // CH
EREM_CH = (E - ECH * NT) // CH
RA = (N // NT) // 8 * 8
NREM = N - RA * NT
BP = 3200
PP = NT * NC * BP


def _elu(x):
    return jnp.where(x > 0, x, jnp.exp(jnp.minimum(x, 0.0)) - 1.0)


def _tc0_body(x_ref, w_ref, b_ref, o_ref):
    o_ref[...] = _elu(
        jnp.dot(x_ref[...], w_ref[...], preferred_element_type=jnp.float32)
        + b_ref[...])


def _node_emb(x, W, b):
    R = 2000
    return pl.pallas_call(
        _tc0_body,
        grid=(N // R,),
        in_specs=[pl.BlockSpec((R, D), lambda i: (i, 0)),
                  pl.BlockSpec((D, D), lambda i: (0, 0)),
                  pl.BlockSpec((1, D), lambda i: (0, 0))],
        out_specs=pl.BlockSpec((R, D), lambda i: (i, 0)),
        out_shape=jax.ShapeDtypeStruct((N, D), jnp.float32),
    )(x, W, b.reshape(1, D))


_RT = 2000
_GT = N // _RT


def _tc12_body(h_ref, w_ref, b_ref, w1_ref, b1_ref, w2_ref, fcw_ref, fcb_ref,
               pw_ref, ne2_ref, uv_ref, sems_v, wp_s):
    i = pl.program_id(0)

    @pl.when(i < _GT)
    def _():
        for t in range(4):
            sem = _elu(
                jnp.dot(h_ref[t], w_ref[t], preferred_element_type=jnp.float32)
                + b_ref[t])
            sems_v[t, pl.ds(i * _RT, _RT)] = sem
            t_ = jnp.tanh(
                jnp.dot(sem, w1_ref[t // 2], preferred_element_type=jnp.float32)
                + b1_ref[t // 2])
            prev = jnp.where(i == 0, 0.0, wp_s[t])
            wp_s[t] = prev + jnp.sum(t_ * w2_ref[t // 2])

    @pl.when(i >= _GT)
    def _():
        sl = pl.ds((i - _GT) * _RT, _RT)
        a0, a1, a2, a3 = (wp_s[0] / N, wp_s[1] / N, wp_s[2] / N, wp_s[3] / N)
        e0 = jnp.exp(a0 - jnp.maximum(a0, a1))
        e1 = jnp.exp(a1 - jnp.maximum(a0, a1))
        e2 = jnp.exp(a2 - jnp.maximum(a2, a3))
        e3 = jnp.exp(a3 - jnp.maximum(a2, a3))
        emb_in = (e0 * sems_v[0, sl] + e1 * sems_v[1, sl]) / (e0 + e1)
        emb_out = (e2 * sems_v[2, sl] + e3 * sems_v[3, sl]) / (e2 + e3)
        ne2 = _elu(
            jnp.dot(emb_in, fcw_ref[0], preferred_element_type=jnp.float32)
            + jnp.dot(emb_out, fcw_ref[1], preferred_element_type=jnp.float32)
            + fcb_ref[...])
        ne2_ref[...] = ne2
        uv_ref[...] = jnp.dot(ne2, pw_ref[...],
                              preferred_element_type=jnp.float32)


def _dense(hsum, Weff, beff, W1, b1, W2, fcWs, fcb, predWcat):
    return pl.pallas_call(
        _tc12_body,
        grid=(2 * _GT,),
        in_specs=[
            pl.BlockSpec((4, _RT, D), lambda i: (0, jnp.minimum(i, _GT - 1), 0)),
            pl.BlockSpec((4, D, D), lambda i: (0, 0, 0)),
            pl.BlockSpec((4, 1, D), lambda i: (0, 0, 0)),
            pl.BlockSpec((2, D, HALF), lambda i: (0, 0, 0)),
            pl.BlockSpec((2, 1, HALF), lambda i: (0, 0, 0)),
            pl.BlockSpec((2, 1, HALF), lambda i: (0, 0, 0)),
            pl.BlockSpec((2, D, D), lambda i: (0, 0, 0)),
            pl.BlockSpec((1, D), lambda i: (0, 0)),
            pl.BlockSpec((D, 4), lambda i: (0, 0)),
        ],
        out_specs=[
            pl.BlockSpec((_RT, D), lambda i: (jnp.maximum(i - _GT, 0), 0)),
            pl.BlockSpec((_RT, 4), lambda i: (jnp.maximum(i - _GT, 0), 0)),
        ],
        out_shape=[jax.ShapeDtypeStruct((N, D), jnp.float32),
                   jax.ShapeDtypeStruct((N, 4), jnp.float32)],
        scratch_shapes=[pltpu.VMEM((4, N, D), jnp.float32),
                        pltpu.SMEM((4,), jnp.float32)],
    )(hsum, Weff, beff, W1, b1, W2, fcWs, fcb, predWcat)


NQ = 6
NRB = 3


def _agg(node_emb, srcs, dsts):
    mesh = plsc.VectorSubcoreMesh(core_axis_name="c", subcore_axis_name="s")

    @functools.partial(
        pl.kernel, mesh=mesh,
        out_type=jax.ShapeDtypeStruct((4, N, D), jnp.float32),
        scratch_types=[
            [pltpu.VMEM((CH,), jnp.int32) for _ in range(NQ)],
            [pltpu.VMEM((CH,), jnp.int32) for _ in range(NQ)],
            [pltpu.VMEM((CH, D), jnp.float32) for _ in range(NRB)],
            pltpu.VMEM_SHARED((N, D), jnp.float32),
            [pltpu.SemaphoreType.DMA for _ in range(NQ)],
            [pltpu.SemaphoreType.DMA for _ in range(NRB)],
            [pltpu.SemaphoreType.DMA for _ in range(NRB)],
        ],
        compiler_params=pltpu.CompilerParams(use_tc_tiling_on_sc=False, needs_layout_passes=False),
    )
    def k(ne_hbm, srcs_hbm, dsts_hbm, out_hbm,
          sidx, didx, rows, acc, isem, gsem, ssem):
        c = lax.axis_index("c")
        s = lax.axis_index("s")

        def idx_cp(base, q):
            return (pltpu.make_async_copy(srcs_hbm.at[pl.ds(base, CH)],
                                          sidx[q], isem[q]),
                    pltpu.make_async_copy(dsts_hbm.at[pl.ds(base, CH)],
                                          didx[q], isem[q]))

        def idx_start(base, q):
            a, bb = idx_cp(base, q)
            a.start()
            bb.start()

        def idx_wait(q):
            a, bb = idx_cp(0, q)
            a.wait()
            bb.wait()

        def gather(q, b):
            return pltpu.make_async_copy(ne_hbm.at[sidx[q]], rows[b], gsem[b])

        def scat(q, b):
            return pltpu.make_async_copy(rows[b], acc.at[didx[q]], ssem[b])

        for j in range(2):
            l = 2 * c + j
            lbase = l * E
            ebase = lbase + s * ECH
            pltpu.sync_copy(ne_hbm.at[pl.ds(s * RA, RA)],
                            acc.at[pl.ds(s * RA, RA)])

            @pl.when(s == 0)
            def _():
                pltpu.sync_copy(ne_hbm.at[pl.ds(RA * NT, NREM)],
                                acc.at[pl.ds(RA * NT, NREM)])

            plsc.subcore_barrier()

            for q in range(4):
                idx_start(ebase + q * CH, q)
            idx_wait(0)
            gather(0, 0).start()
            idx_wait(1)
            gather(1, 1).start()

            def body(k_, _):
                for u in range(NQ):
                    g = k_ * NQ + u
                    b = u % NRB
                    gather(u, b).wait()

                    @pl.when(g >= 1)
                    def _():
                        scat((u + 2) % NQ, (u + 2) % NRB).wait()

                    @pl.when(g + 4 < NCH)
                    def _():
                        idx_start(ebase + (g + 4) * CH, (u + 4) % NQ)

                    @pl.when(g + 2 < NCH)
                    def _():
                        idx_wait((u + 2) % NQ)
                        gather((u + 2) % NQ, (u + 2) % NRB).start()

                    scat(u, b).start(add=True)
                return ()

            lax.fori_loop(0, NCH // NQ, body, ())
            scat(5, 2).wait()

            @pl.when(s < EREM_CH)
            def _():
                rbase = lbase + ECH * NT + s * CH
                idx_start(rbase, 0)
                idx_wait(0)
                pltpu.async_copy(ne_hbm.at[sidx[0]], rows[0], gsem[0]).wait()
                pltpu.sync_copy(rows[0], acc.at[didx[0]], add=True)

            plsc.subcore_barrier()
            pltpu.sync_copy(acc.at[pl.ds(s * RA, RA)],
                            out_hbm.at[l, pl.ds(s * RA, RA)])

            @pl.when(s == 0)
            def _():
                pltpu.sync_copy(acc.at[pl.ds(RA * NT, NREM)],
                                out_hbm.at[l, pl.ds(RA * NT, NREM)])

            plsc.subcore_barrier()

    return k(node_emb, srcs, dsts)


def _pair_logits(UVflat, eidx_flat):
    mesh = plsc.VectorSubcoreMesh(core_axis_name="c", subcore_axis_name="s")

    @functools.partial(
        pl.kernel, mesh=mesh,
        out_type=jax.ShapeDtypeStruct((2 * PP,), jnp.float32),
        scratch_types=[
            pltpu.VMEM((4 * N,), jnp.float32),
            pltpu.VMEM((BP,), jnp.int32),
            pltpu.VMEM((BP,), jnp.int32),
            pltpu.VMEM((BP,), jnp.float32),
            pltpu.VMEM((BP,), jnp.float32),
        ],
        compiler_params=pltpu.CompilerParams(use_tc_tiling_on_sc=False, needs_layout_passes=False),
    )
    def k(uv_hbm, eidx_hbm, out_hbm, uv_v, i0_v, i1_v, o0_v, o1_v):
        c = lax.axis_index("c")
        s = lax.axis_index("s")
        wid = s * NC + c
        base = wid * BP
        pltpu.sync_copy(uv_hbm, uv_v)
        pltpu.sync_copy(eidx_hbm.at[pl.ds(base, BP)], i0_v)
        pltpu.sync_copy(eidx_hbm.at[pl.ds(PP + base, BP)], i1_v)

        def body(g, _):
            sl = pl.ds(g * 16, 16)
            i0 = i0_v[sl] * 4
            i1 = i1_v[sl] * 4
            u0 = plsc.load_gather(uv_v, [i0])
            v0 = plsc.load_gather(uv_v, [i1 + 2])
            u1 = plsc.load_gather(uv_v, [i0 + 1])
            v1 = plsc.load_gather(uv_v, [i1 + 3])
            o0_v[sl] = u0 + v0
            o1_v[sl] = u1 + v1
            return ()

        lax.fori_loop(0, BP // 16, body, ())
        pltpu.sync_copy(o0_v, out_hbm.at[pl.ds(base, BP)])
        pltpu.sync_copy(o1_v, out_hbm.at[pl.ds(PP + base, BP)])

    return k(UVflat, eidx_flat)


def _edge_cs(edge_feat, fc_edge):
    emb = jax.nn.elu(jnp.squeeze(jnp.matmul(edge_feat, fc_edge), axis=1))
    r1, r2 = jnp.split(emb, 2, axis=-1)
    nrm = jnp.maximum(jnp.sqrt(r1 * r1 + r2 * r2), 1e-12)
    return r1 / nrm, r2 / nrm


def _step_mat(c, s, tag):
    if tag == "in":
        return (c, -s, s * c, c - s * s)
    return (c, s, -c * s, c - s * s)


def _compose(mb, ma):
    b00, b01, b10, b11 = mb
    a00, a01, a10, a11 = ma
    return (b00 * a00 + b01 * a10, b00 * a01 + b01 * a11,
            b10 * a00 + b11 * a10, b10 * a01 + b11 * a11)


def _path_mat(c, s, path, tag):
    mats = {e: _step_mat(c[e - 1], s[e - 1], tag) for e in (1, 2)}
    seq = path if tag == "in" else list(reversed(path))
    M = mats[seq[0]]
    for e in seq[1:]:
        M = _compose(mats[e], M)
    return M


def _weff(M, W):
    m00, m01, m10, m11 = M
    Wt, Wb = W[:HALF], W[HALF:]
    we = m00[:, None] * Wt + m10[:, None] * Wb
    wo = m01[:, None] * Wt + m11[:, None] * Wb
    return jnp.stack([we, wo], axis=1).reshape(D, D)


def kernel(node_feat, fc_node_W, fc_node_b, edge_feat_in, edge_feat_out,
           fc_edge_in, fc_edge_out, gin_in_W, gin_in_b, sa_in_W1, sa_in_b1,
           sa_in_W2, gin_out_W, gin_out_b, sa_out_W1, sa_out_b1, sa_out_W2,
           fc_W, fc_b, pred_W, edge_index_in_0, edge_index_in_1,
           edge_index_out_0, edge_index_out_1, edge_indices):
    node_emb = _node_emb(node_feat, fc_node_W, fc_node_b)

    cin, sin = _edge_cs(edge_feat_in, fc_edge_in)
    cout, sout = _edge_cs(edge_feat_out, fc_edge_out)
    Ms = [_path_mat(cin, sin, [1, 2], "in"),
          _path_mat(cin, sin, [2, 1], "in"),
          _path_mat(cout, sout, [1, 2], "out"),
          _path_mat(cout, sout, [2, 1], "out")]
    Gw = [gin_in_W[0], gin_in_W[1], gin_out_W[0], gin_out_W[1]]
    Weff = jnp.stack([_weff(Ms[i], Gw[i]) for i in range(4)])
    beff = jnp.stack([gin_in_b, gin_out_b]).reshape(4, 1, D)
    W1 = jnp.stack([sa_in_W1, sa_out_W1])
    b1 = jnp.stack([sa_in_b1, sa_out_b1]).reshape(2, 1, HALF)
    W2 = jnp.stack([sa_in_W2, sa_out_W2]).reshape(2, 1, HALF)

    srcs = jnp.concatenate([edge_index_in_0[0], edge_index_in_1[0],
                            edge_index_out_0[0], edge_index_out_1[0]])
    dsts = jnp.concatenate([edge_index_in_0[1], edge_index_in_1[1],
                            edge_index_out_0[1], edge_index_out_1[1]])

    hsum = _agg(node_emb, srcs, dsts)

    fcWs = jnp.stack([fc_W[:D], fc_W[D:]])
    predWcat = jnp.concatenate([pred_W[:D], pred_W[D:]], axis=1)
    node_emb2, UV = _dense(hsum, Weff, beff, W1, b1, W2, fcWs,
                           fc_b.reshape(1, D), predWcat)

    eidx_flat = jnp.pad(edge_indices, ((0, PP - P), (0, 0))).T.reshape(-1)
    lg = _pair_logits(UV.reshape(-1), eidx_flat)
    logits = lg.reshape(2, PP).T[:P]
    return (node_emb2, logits)


# speedup vs baseline: 1.6057x; 1.1077x over previous
import functools

import jax
import jax.numpy as jnp
from jax import lax
from jax.experimental import pallas as pl
from jax.experimental.pallas import tpu as pltpu
from jax.experimental.pallas import tpu_sc as plsc

N = 10000
D = 128
HALF = 64
E = 320000
P = 100000

NT = 16
NC = 2
CH = 128
ECH = (E // NT) // CH * CH
NCH = ECH // CH
EREM_CH = (E - ECH * NT) // CH
RA = (N // NT) // 8 * 8
NREM = N - RA * NT
BP = 3200
PP = NT * NC * BP


def _elu(x):
    return jnp.where(x > 0, x, jnp.exp(jnp.minimum(x, 0.0)) - 1.0)


def _tc0_body(x_ref, w_ref, b_ref, o_ref):
    o_ref[...] = _elu(
        jnp.dot(x_ref[...], w_ref[...], preferred_element_type=jnp.float32)
        + b_ref[...])


def _node_emb(x, W, b):
    R = 2000
    return pl.pallas_call(
        _tc0_body,
        grid=(N // R,),
        in_specs=[pl.BlockSpec((R, D), lambda i: (i, 0)),
                  pl.BlockSpec((D, D), lambda i: (0, 0)),
                  pl.BlockSpec((1, D), lambda i: (0, 0))],
        out_specs=pl.BlockSpec((R, D), lambda i: (i, 0)),
        out_shape=jax.ShapeDtypeStruct((N, D), jnp.float32),
    )(x, W, b.reshape(1, D))


_RT = 2000
_GT = N // _RT


def _tc12_body(h_ref, w_ref, b_ref, w1_ref, b1_ref, w2_ref, fcw_ref, fcb_ref,
               pw_ref, ne2_ref, uv_ref, sems_v, wp_s):
    i = pl.program_id(0)

    @pl.when(i < _GT)
    def _():
        for t in range(4):
            sem = _elu(
                jnp.dot(h_ref[t], w_ref[t], preferred_element_type=jnp.float32)
                + b_ref[t])
            sems_v[t, pl.ds(i * _RT, _RT)] = sem
            t_ = jnp.tanh(
                jnp.dot(sem, w1_ref[t // 2], preferred_element_type=jnp.float32)
                + b1_ref[t // 2])
            prev = jnp.where(i == 0, 0.0, wp_s[t])
            wp_s[t] = prev + jnp.sum(t_ * w2_ref[t // 2])

    @pl.when(i >= _GT)
    def _():
        sl = pl.ds((i - _GT) * _RT, _RT)
        a0, a1, a2, a3 = (wp_s[0] / N, wp_s[1] / N, wp_s[2] / N, wp_s[3] / N)
        e0 = jnp.exp(a0 - jnp.maximum(a0, a1))
        e1 = jnp.exp(a1 - jnp.maximum(a0, a1))
        e2 = jnp.exp(a2 - jnp.maximum(a2, a3))
        e3 = jnp.exp(a3 - jnp.maximum(a2, a3))
        emb_in = (e0 * sems_v[0, sl] + e1 * sems_v[1, sl]) / (e0 + e1)
        emb_out = (e2 * sems_v[2, sl] + e3 * sems_v[3, sl]) / (e2 + e3)
        ne2 = _elu(
            jnp.dot(emb_in, fcw_ref[0], preferred_element_type=jnp.float32)
            + jnp.dot(emb_out, fcw_ref[1], preferred_element_type=jnp.float32)
            + fcb_ref[...])
        ne2_ref[...] = ne2
        uv_ref[...] = jnp.dot(ne2, pw_ref[...],
                              preferred_element_type=jnp.float32)


def _dense(hsum, Weff, beff, W1, b1, W2, fcWs, fcb, predWcat):
    return pl.pallas_call(
        _tc12_body,
        grid=(2 * _GT,),
        in_specs=[
            pl.BlockSpec((4, _RT, D), lambda i: (0, jnp.minimum(i, _GT - 1), 0)),
            pl.BlockSpec((4, D, D), lambda i: (0, 0, 0)),
            pl.BlockSpec((4, 1, D), lambda i: (0, 0, 0)),
            pl.BlockSpec((2, D, HALF), lambda i: (0, 0, 0)),
            pl.BlockSpec((2, 1, HALF), lambda i: (0, 0, 0)),
            pl.BlockSpec((2, 1, HALF), lambda i: (0, 0, 0)),
            pl.BlockSpec((2, D, D), lambda i: (0, 0, 0)),
            pl.BlockSpec((1, D), lambda i: (0, 0)),
            pl.BlockSpec((D, 4), lambda i: (0, 0)),
        ],
        out_specs=[
            pl.BlockSpec((_RT, D), lambda i: (jnp.maximum(i - _GT, 0), 0)),
            pl.BlockSpec((_RT, 4), lambda i: (jnp.maximum(i - _GT, 0), 0)),
        ],
        out_shape=[jax.ShapeDtypeStruct((N, D), jnp.float32),
                   jax.ShapeDtypeStruct((N, 4), jnp.float32)],
        scratch_shapes=[pltpu.VMEM((4, N, D), jnp.float32),
                        pltpu.SMEM((4,), jnp.float32)],
    )(hsum, Weff, beff, W1, b1, W2, fcWs, fcb, predWcat)


NQ = 6
NRB = 3


def _agg(node_emb, ei_in0, ei_in1, ei_out0, ei_out1):
    mesh = plsc.VectorSubcoreMesh(core_axis_name="c", subcore_axis_name="s")

    @functools.partial(
        pl.kernel, mesh=mesh,
        out_type=jax.ShapeDtypeStruct((4, N, D), jnp.float32),
        scratch_types=[
            [pltpu.VMEM((CH,), jnp.int32) for _ in range(NQ)],
            [pltpu.VMEM((CH,), jnp.int32) for _ in range(NQ)],
            [pltpu.VMEM((CH, D), jnp.float32) for _ in range(NRB)],
            pltpu.VMEM_SHARED((N, D), jnp.float32),
            [pltpu.SemaphoreType.DMA for _ in range(NQ)],
            [pltpu.SemaphoreType.DMA for _ in range(NRB)],
            [pltpu.SemaphoreType.DMA for _ in range(NRB)],
        ],
        compiler_params=pltpu.CompilerParams(use_tc_tiling_on_sc=False, needs_layout_passes=False),
    )
    def k(ne_hbm, in0_hbm, in1_hbm, out0_hbm, out1_hbm, out_hbm,
          sidx, didx, rows, acc, isem, gsem, ssem):
        c = lax.axis_index("c")
        s = lax.axis_index("s")

        def gather(q, b):
            return pltpu.make_async_copy(ne_hbm.at[sidx[q]], rows[b], gsem[b])

        def scat(q, b):
            return pltpu.make_async_copy(rows[b], acc.at[didx[q]], ssem[b])

        def process(ei_hbm):
            ebase = s * ECH

            def idx_cp(base, q):
                return (pltpu.make_async_copy(ei_hbm.at[0, pl.ds(base, CH)],
                                              sidx[q], isem[q]),
                        pltpu.make_async_copy(ei_hbm.at[1, pl.ds(base, CH)],
                                              didx[q], isem[q]))

            def idx_start(base, q):
                a, bb = idx_cp(base, q)
                a.start()
                bb.start()

            def idx_wait(q):
                a, bb = idx_cp(0, q)
                a.wait()
                bb.wait()

            for q in range(4):
                idx_start(ebase + q * CH, q)
            idx_wait(0)
            gather(0, 0).start()
            idx_wait(1)
            gather(1, 1).start()

            def body(k_, _):
                for u in range(NQ):
                    g = k_ * NQ + u
                    b = u % NRB
                    gather(u, b).wait()

                    @pl.when(g >= 1)
                    def _():
                        scat((u + 2) % NQ, (u + 2) % NRB).wait()

                    @pl.when(g + 4 < NCH)
                    def _():
                        idx_start(ebase + (g + 4) * CH, (u + 4) % NQ)

                    @pl.when(g + 2 < NCH)
                    def _():
                        idx_wait((u + 2) % NQ)
                        gather((u + 2) % NQ, (u + 2) % NRB).start()

                    scat(u, b).start(add=True)
                return ()

            lax.fori_loop(0, NCH // NQ, body, ())
            scat(5, 2).wait()

            @pl.when(s < EREM_CH)
            def _():
                rbase = ECH * NT + s * CH
                idx_start(rbase, 0)
                idx_wait(0)
                pltpu.async_copy(ne_hbm.at[sidx[0]], rows[0], gsem[0]).wait()
                pltpu.sync_copy(rows[0], acc.at[didx[0]], add=True)

        def writeback(lidx):
            pltpu.sync_copy(acc.at[pl.ds(s * RA, RA)],
                            out_hbm.at[lidx, pl.ds(s * RA, RA)])

            @pl.when(s == 0)
            def _():
                pltpu.sync_copy(acc.at[pl.ds(RA * NT, NREM)],
                                out_hbm.at[lidx, pl.ds(RA * NT, NREM)])

        for j in range(2):
            pltpu.sync_copy(ne_hbm.at[pl.ds(s * RA, RA)],
                            acc.at[pl.ds(s * RA, RA)])

            @pl.when(s == 0)
            def _():
                pltpu.sync_copy(ne_hbm.at[pl.ds(RA * NT, NREM)],
                                acc.at[pl.ds(RA * NT, NREM)])

            plsc.subcore_barrier()

            @pl.when(c == 0)
            def _():
                process(in0_hbm if j == 0 else in1_hbm)

            @pl.when(c == 1)
            def _():
                process(out0_hbm if j == 0 else out1_hbm)

            plsc.subcore_barrier()

            @pl.when(c == 0)
            def _():
                writeback(j)

            @pl.when(c == 1)
            def _():
                writeback(2 + j)

            plsc.subcore_barrier()

    return k(node_emb, ei_in0, ei_in1, ei_out0, ei_out1)


def _pair_logits(UVflat, eidx_flat):
    mesh = plsc.VectorSubcoreMesh(core_axis_name="c", subcore_axis_name="s")

    @functools.partial(
        pl.kernel, mesh=mesh,
        out_type=jax.ShapeDtypeStruct((2 * PP,), jnp.float32),
        scratch_types=[
            pltpu.VMEM((4 * N,), jnp.float32),
            pltpu.VMEM((BP,), jnp.int32),
            pltpu.VMEM((BP,), jnp.int32),
            pltpu.VMEM((BP,), jnp.float32),
            pltpu.VMEM((BP,), jnp.float32),
        ],
        compiler_params=pltpu.CompilerParams(use_tc_tiling_on_sc=False, needs_layout_passes=False),
    )
    def k(uv_hbm, eidx_hbm, out_hbm, uv_v, i0_v, i1_v, o0_v, o1_v):
        c = lax.axis_index("c")
        s = lax.axis_index("s")
        wid = s * NC + c
        base = wid * BP
        pltpu.sync_copy(uv_hbm, uv_v)
        pltpu.sync_copy(eidx_hbm.at[pl.ds(base, BP)], i0_v)
        pltpu.sync_copy(eidx_hbm.at[pl.ds(PP + base, BP)], i1_v)

        def body(g, _):
            sl = pl.ds(g * 16, 16)
            i0 = i0_v[sl] * 4
            i1 = i1_v[sl] * 4
            u0 = plsc.load_gather(uv_v, [i0])
            v0 = plsc.load_gather(uv_v, [i1 + 2])
            u1 = plsc.load_gather(uv_v, [i0 + 1])
            v1 = plsc.load_gather(uv_v, [i1 + 3])
            o0_v[sl] = u0 + v0
            o1_v[sl] = u1 + v1
            return ()

        lax.fori_loop(0, BP // 16, body, ())
        pltpu.sync_copy(o0_v, out_hbm.at[pl.ds(base, BP)])
        pltpu.sync_copy(o1_v, out_hbm.at[pl.ds(PP + base, BP)])

    return k(UVflat, eidx_flat)


def _edge_cs(edge_feat, fc_edge):
    emb = jax.nn.elu(jnp.squeeze(jnp.matmul(edge_feat, fc_edge), axis=1))
    r1, r2 = jnp.split(emb, 2, axis=-1)
    nrm = jnp.maximum(jnp.sqrt(r1 * r1 + r2 * r2), 1e-12)
    return r1 / nrm, r2 / nrm


def _step_mat(c, s, tag):
    if tag == "in":
        return (c, -s, s * c, c - s * s)
    return (c, s, -c * s, c - s * s)


def _compose(mb, ma):
    b00, b01, b10, b11 = mb
    a00, a01, a10, a11 = ma
    return (b00 * a00 + b01 * a10, b00 * a01 + b01 * a11,
            b10 * a00 + b11 * a10, b10 * a01 + b11 * a11)


def _path_mat(c, s, path, tag):
    mats = {e: _step_mat(c[e - 1], s[e - 1], tag) for e in (1, 2)}
    seq = path if tag == "in" else list(reversed(path))
    M = mats[seq[0]]
    for e in seq[1:]:
        M = _compose(mats[e], M)
    return M


def _weff(M, W):
    m00, m01, m10, m11 = M
    Wt, Wb = W[:HALF], W[HALF:]
    we = m00[:, None] * Wt + m10[:, None] * Wb
    wo = m01[:, None] * Wt + m11[:, None] * Wb
    return jnp.stack([we, wo], axis=1).reshape(D, D)


def kernel(node_feat, fc_node_W, fc_node_b, edge_feat_in, edge_feat_out,
           fc_edge_in, fc_edge_out, gin_in_W, gin_in_b, sa_in_W1, sa_in_b1,
           sa_in_W2, gin_out_W, gin_out_b, sa_out_W1, sa_out_b1, sa_out_W2,
           fc_W, fc_b, pred_W, edge_index_in_0, edge_index_in_1,
           edge_index_out_0, edge_index_out_1, edge_indices):
    node_emb = _node_emb(node_feat, fc_node_W, fc_node_b)

    cin, sin = _edge_cs(edge_feat_in, fc_edge_in)
    cout, sout = _edge_cs(edge_feat_out, fc_edge_out)
    Ms = [_path_mat(cin, sin, [1, 2], "in"),
          _path_mat(cin, sin, [2, 1], "in"),
          _path_mat(cout, sout, [1, 2], "out"),
          _path_mat(cout, sout, [2, 1], "out")]
    Gw = [gin_in_W[0], gin_in_W[1], gin_out_W[0], gin_out_W[1]]
    Weff = jnp.stack([_weff(Ms[i], Gw[i]) for i in range(4)])
    beff = jnp.stack([gin_in_b, gin_out_b]).reshape(4, 1, D)
    W1 = jnp.stack([sa_in_W1, sa_out_W1])
    b1 = jnp.stack([sa_in_b1, sa_out_b1]).reshape(2, 1, HALF)
    W2 = jnp.stack([sa_in_W2, sa_out_W2]).reshape(2, 1, HALF)

    hsum = _agg(node_emb, edge_index_in_0, edge_index_in_1,
                edge_index_out_0, edge_index_out_1)

    fcWs = jnp.stack([fc_W[:D], fc_W[D:]])
    predWcat = jnp.concatenate([pred_W[:D], pred_W[D:]], axis=1)
    node_emb2, UV = _dense(hsum, Weff, beff, W1, b1, W2, fcWs,
                           fc_b.reshape(1, D), predWcat)

    eidx_flat = jnp.pad(edge_indices, ((0, PP - P), (0, 0))).T.reshape(-1)
    lg = _pair_logits(UV.reshape(-1), eidx_flat)
    logits = lg.reshape(2, PP).T[:P]
    return (node_emb2, logits)
